# baseline (speedup 1.0000x reference)
import functools

import jax
import jax.numpy as jnp
from jax import lax
from jax.experimental import pallas as pl
from jax.experimental.pallas import tpu as pltpu

try:
    from jax.experimental.pallas import tpu_sc as plsc
except ImportError:
    plsc = None

_K = 8
_NEG = -1e30
_FBIG = 3e38


def _topk_body(q_ref, k_ref, vals_ref, idx_ref, *, n_keys, tile):
    t = pl.program_id(0)

    q = q_ref[...]
    qn = q / jnp.maximum(jnp.sqrt(jnp.sum(q * q, axis=1, keepdims=True)), 1e-8)
    k = k_ref[...]
    kn = k / jnp.maximum(jnp.sqrt(jnp.sum(k * k, axis=1, keepdims=True)), 1e-8)

    s = lax.dot_general(qn, kn, (((1,), (1,)), ((), ())),
                        preferred_element_type=jnp.float32)
    gcol = (jnp.float32(t * tile)
            + lax.broadcasted_iota(jnp.int32, s.shape, 1).astype(jnp.float32))
    s = jnp.where(gcol < jnp.float32(n_keys), s, _NEG)

    nch = tile // 128
    cv = [s[:, j * 128:(j + 1) * 128] for j in range(nch)]
    ci_ = [gcol[:, j * 128:(j + 1) * 128] for j in range(nch)]

    def cswap(j, k):
        a, b = cv[j], cv[k]
        keep = a >= b
        cv[j], cv[k] = jnp.maximum(a, b), jnp.minimum(a, b)
        ci_[j], ci_[k] = (jnp.where(keep, ci_[j], ci_[k]),
                          jnp.where(keep, ci_[k], ci_[j]))

    for j, k in ((0, 1), (2, 3), (4, 5), (6, 7),
                 (0, 2), (1, 3), (4, 6), (5, 7),
                 (1, 2), (5, 6),
                 (0, 4), (1, 5), (2, 6), (3, 7),
                 (2, 4), (3, 5),
                 (1, 2), (3, 4), (5, 6)):
        cswap(j, k)

    tv, ti = [], []
    for _ in range(_K):
        m = jnp.max(cv[0], axis=1, keepdims=True)
        cidx = jnp.min(jnp.where(cv[0] >= m, ci_[0], _FBIG),
                       axis=1, keepdims=True)
        tv.append(m)
        ti.append(cidx)
        onehot = ci_[0] == cidx
        for j in range(nch - 1):
            cv[j] = jnp.where(onehot, cv[j + 1], cv[j])
            ci_[j] = jnp.where(onehot, ci_[j + 1], ci_[j])
        cv[nch - 1] = jnp.where(onehot, _NEG, cv[nch - 1])
    vals_ref[0] = jnp.concatenate(tv, axis=1)
    idx_ref[0] = jnp.concatenate(ti, axis=1)


def _merge_body(vals_ref, idx_ref, ov_ref, oi_ref):
    mv = vals_ref[...]
    mi = idx_ref[...]
    nv, ni = [], []
    for _ in range(_K):
        m = jnp.max(mv, axis=1, keepdims=True)
        ci = jnp.min(jnp.where(mv >= m, mi, _FBIG), axis=1, keepdims=True)
        nv.append(m)
        ni.append(ci)
        mv = jnp.where(mi == ci, _NEG, mv)
    ov_ref[...] = jnp.concatenate(nv, axis=1)
    oi_ref[...] = jnp.concatenate(ni, axis=1).astype(jnp.int32)


def _run_topk(queries, keys):
    q_n, d = queries.shape
    n_keys = keys.shape[0]
    tile = 1024
    nt = (n_keys + tile - 1) // tile
    cand_vals, cand_idx = pl.pallas_call(
        functools.partial(_topk_body, n_keys=n_keys, tile=tile),
        grid=(nt,),
        in_specs=[
            pl.BlockSpec((q_n, d), lambda t: (0, 0)),
            pl.BlockSpec((tile, d), lambda t: (t, 0)),
        ],
        out_specs=[
            pl.BlockSpec((1, q_n, _K), lambda t: (t, 0, 0)),
            pl.BlockSpec((1, q_n, _K), lambda t: (t, 0, 0)),
        ],
        out_shape=[
            jax.ShapeDtypeStruct((nt, q_n, _K), jnp.float32),
            jax.ShapeDtypeStruct((nt, q_n, _K), jnp.float32),
        ],
        compiler_params=pltpu.CompilerParams(
            dimension_semantics=("arbitrary",)),
    )(queries, keys)
    cand_vals = jnp.swapaxes(cand_vals, 0, 1).reshape(q_n, nt * _K)
    cand_idx = jnp.swapaxes(cand_idx, 0, 1).reshape(q_n, nt * _K)
    return pl.pallas_call(
        _merge_body,
        out_shape=[
            jax.ShapeDtypeStruct((q_n, _K), jnp.float32),
            jax.ShapeDtypeStruct((q_n, _K), jnp.int32),
        ],
    )(cand_vals, cand_idx)


def _perm_lanes(w, idx):
    lanes = w.shape[0]
    return lax.gather(
        w,
        idx.reshape(lanes, 1),
        lax.GatherDimensionNumbers(
            offset_dims=(), collapsed_slice_dims=(0,), start_index_map=(0,)),
        slice_sizes=(1,),
        mode=lax.GatherScatterMode.PROMISE_IN_BOUNDS,
    )


def _bcast_lane(w, j, lanes):
    return _perm_lanes(w, jnp.full((lanes,), j, jnp.int32))


def _make_aggregate_sc(q_n, d, n_cores, n_subcores, lanes):
    qpw = q_n // (n_cores * n_subcores)
    pairs = qpw // 2
    mesh = plsc.VectorSubcoreMesh(core_axis_name="c", subcore_axis_name="s")
    nvec = d // lanes

    @functools.partial(
        pl.kernel,
        mesh=mesh,
        out_type=jax.ShapeDtypeStruct((q_n, d), jnp.float32),
        scratch_types=[
            pltpu.VMEM((qpw * _K,), jnp.int32),
            pltpu.VMEM((qpw * _K,), jnp.float32),
            pltpu.VMEM((16,), jnp.float32),
            pltpu.VMEM((16, d), jnp.float32),
            pltpu.VMEM((qpw, d), jnp.float32),
            pltpu.SemaphoreType.DMA,
        ],
    )
    def agg(idx_hbm, vals_hbm, values_hbm, conf_hbm, out_hbm,
            idx_v, sims_v, conf_v, rows_v, out_v, sem):
        wid = lax.axis_index("s") * n_cores + lax.axis_index("c")
        base = wid * (qpw * _K)
        pltpu.sync_copy(idx_hbm.at[pl.ds(base, qpw * _K)], idx_v)
        pltpu.sync_copy(vals_hbm.at[pl.ds(base, qpw * _K)], sims_v)

        for p in range(pairs):
            sl = pl.ds(p * 16, 16)
            idx16 = idx_v[sl]
            sim16 = sims_v[sl]
            pltpu.async_copy(conf_hbm.at[idx16], conf_v, sem).wait()
            pltpu.async_copy(values_hbm.at[idx16], rows_v, sem).wait()
            conf16 = conf_v[...]
            w = jnp.maximum(conf16, jnp.float32(1e-4)) * sim16
            lane = lax.iota(jnp.int32, 16)
            gsum = w
            for sh in (4, 2, 1):
                gsum = gsum + _perm_lanes(gsum, lane ^ sh)
            wn = w / (gsum + jnp.float32(1e-8))
            wbc = [_bcast_lane(wn, j, lanes) for j in range(16)]
            for c in range(nvec):
                csl = pl.ds(c * lanes, lanes)
                acc_a = jnp.zeros((lanes,), jnp.float32)
                acc_b = jnp.zeros((lanes,), jnp.float32)
                for j in range(_K):
                    acc_a = acc_a + wbc[j] * rows_v[j, csl]
                for j in range(_K, 16):
                    acc_b = acc_b + wbc[j] * rows_v[j, csl]
                out_v[2 * p, csl] = acc_a
                out_v[2 * p + 1, csl] = acc_b

        pltpu.sync_copy(out_v, out_hbm.at[pl.ds(wid * qpw, qpw)])

    return agg


def kernel(queries, keys, values, confidence, top_k):
    q_n, d = queries.shape
    top_vals, top_idx = _run_topk(queries, keys)

    info = plsc.get_sparse_core_info()
    agg = _make_aggregate_sc(q_n, d, info.num_cores, info.num_subcores,
                             info.num_lanes)
    out = agg(top_idx.reshape(-1), top_vals.reshape(-1), values, confidence)
    return out


# speedup vs baseline: 1.1470x; 1.1470x over previous
import functools

import jax
import jax.numpy as jnp
from jax import lax
from jax.experimental import pallas as pl
from jax.experimental.pallas import tpu as pltpu

try:
    from jax.experimental.pallas import tpu_sc as plsc
except ImportError:
    plsc = None

_K = 8
_NEG = -1e30
_FBIG = 3e38


def _topk_body(q_ref, k_ref, vals_ref, idx_ref, *, n_keys, tile):
    t = pl.program_id(0)

    q = q_ref[...]
    qn = q / jnp.maximum(jnp.sqrt(jnp.sum(q * q, axis=1, keepdims=True)), 1e-8)
    k = k_ref[...]
    kn = k / jnp.maximum(jnp.sqrt(jnp.sum(k * k, axis=1, keepdims=True)), 1e-8)

    s = lax.dot_general(qn, kn, (((1,), (1,)), ((), ())),
                        preferred_element_type=jnp.float32)
    gcol = (jnp.float32(t * tile)
            + lax.broadcasted_iota(jnp.int32, s.shape, 1).astype(jnp.float32))
    s = jnp.where(gcol < jnp.float32(n_keys), s, _NEG)

    tv, ti = [], []
    for _ in range(_K):
        m = jnp.max(s, axis=1, keepdims=True)
        ci = jnp.min(jnp.where(s >= m, gcol, _FBIG), axis=1, keepdims=True)
        tv.append(m)
        ti.append(ci)
        s = jnp.where(gcol == ci, _NEG, s)
    vals_ref[0] = jnp.concatenate(tv, axis=1).T
    idx_ref[0] = jnp.concatenate(ti, axis=1).T


def _merge_body(vals_ref, idx_ref, ov_ref, oi_ref):
    mv = vals_ref[...]
    mi = idx_ref[...]
    nv, ni = [], []
    for _ in range(_K):
        m = jnp.max(mv, axis=0, keepdims=True)
        ci = jnp.min(jnp.where(mv >= m, mi, _FBIG), axis=0, keepdims=True)
        nv.append(m)
        ni.append(ci)
        mv = jnp.where(mi == ci, _NEG, mv)
    ov_ref[...] = jnp.concatenate(nv, axis=0)
    oi_ref[...] = jnp.concatenate(ni, axis=0).astype(jnp.int32)


def _run_topk(queries, keys):
    q_n, d = queries.shape
    n_keys = keys.shape[0]
    tile = 1024
    nt = (n_keys + tile - 1) // tile
    cand_vals, cand_idx = pl.pallas_call(
        functools.partial(_topk_body, n_keys=n_keys, tile=tile),
        grid=(nt,),
        in_specs=[
            pl.BlockSpec((q_n, d), lambda t: (0, 0)),
            pl.BlockSpec((tile, d), lambda t: (t, 0)),
        ],
        out_specs=[
            pl.BlockSpec((1, _K, q_n), lambda t: (t, 0, 0)),
            pl.BlockSpec((1, _K, q_n), lambda t: (t, 0, 0)),
        ],
        out_shape=[
            jax.ShapeDtypeStruct((nt, _K, q_n), jnp.float32),
            jax.ShapeDtypeStruct((nt, _K, q_n), jnp.float32),
        ],
        compiler_params=pltpu.CompilerParams(
            dimension_semantics=("arbitrary",)),
    )(queries, keys)
    cand_vals = cand_vals.reshape(nt * _K, q_n)
    cand_idx = cand_idx.reshape(nt * _K, q_n)
    tv, ti = pl.pallas_call(
        _merge_body,
        out_shape=[
            jax.ShapeDtypeStruct((_K, q_n), jnp.float32),
            jax.ShapeDtypeStruct((_K, q_n), jnp.int32),
        ],
    )(cand_vals, cand_idx)
    return tv.T, ti.T


def _perm_lanes(w, idx):
    lanes = w.shape[0]
    return lax.gather(
        w,
        idx.reshape(lanes, 1),
        lax.GatherDimensionNumbers(
            offset_dims=(), collapsed_slice_dims=(0,), start_index_map=(0,)),
        slice_sizes=(1,),
        mode=lax.GatherScatterMode.PROMISE_IN_BOUNDS,
    )


def _bcast_lane(w, j, lanes):
    return _perm_lanes(w, jnp.full((lanes,), j, jnp.int32))


def _make_aggregate_sc(q_n, d, n_cores, n_subcores, lanes):
    qpw = q_n // (n_cores * n_subcores)
    pairs = qpw // 2
    mesh = plsc.VectorSubcoreMesh(core_axis_name="c", subcore_axis_name="s")
    nvec = d // lanes

    @functools.partial(
        pl.kernel,
        mesh=mesh,
        out_type=jax.ShapeDtypeStruct((q_n, d), jnp.float32),
        scratch_types=[
            pltpu.VMEM((qpw * _K,), jnp.int32),
            pltpu.VMEM((qpw * _K,), jnp.float32),
            pltpu.VMEM((16,), jnp.float32),
            pltpu.VMEM((16, d), jnp.float32),
            pltpu.VMEM((qpw, d), jnp.float32),
            pltpu.SemaphoreType.DMA,
        ],
    )
    def agg(idx_hbm, vals_hbm, values_hbm, conf_hbm, out_hbm,
            idx_v, sims_v, conf_v, rows_v, out_v, sem):
        wid = lax.axis_index("s") * n_cores + lax.axis_index("c")
        base = wid * (qpw * _K)
        pltpu.sync_copy(idx_hbm.at[pl.ds(base, qpw * _K)], idx_v)
        pltpu.sync_copy(vals_hbm.at[pl.ds(base, qpw * _K)], sims_v)

        for p in range(pairs):
            sl = pl.ds(p * 16, 16)
            idx16 = idx_v[sl]
            sim16 = sims_v[sl]
            pltpu.async_copy(conf_hbm.at[idx16], conf_v, sem).wait()
            pltpu.async_copy(values_hbm.at[idx16], rows_v, sem).wait()
            conf16 = conf_v[...]
            w = jnp.maximum(conf16, jnp.float32(1e-4)) * sim16
            lane = lax.iota(jnp.int32, 16)
            gsum = w
            for sh in (4, 2, 1):
                gsum = gsum + _perm_lanes(gsum, lane ^ sh)
            wn = w / (gsum + jnp.float32(1e-8))
            wbc = [_bcast_lane(wn, j, lanes) for j in range(16)]
            for c in range(nvec):
                csl = pl.ds(c * lanes, lanes)
                acc_a = jnp.zeros((lanes,), jnp.float32)
                acc_b = jnp.zeros((lanes,), jnp.float32)
                for j in range(_K):
                    acc_a = acc_a + wbc[j] * rows_v[j, csl]
                for j in range(_K, 16):
                    acc_b = acc_b + wbc[j] * rows_v[j, csl]
                out_v[2 * p, csl] = acc_a
                out_v[2 * p + 1, csl] = acc_b

        pltpu.sync_copy(out_v, out_hbm.at[pl.ds(wid * qpw, qpw)])

    return agg


def kernel(queries, keys, values, confidence, top_k):
    q_n, d = queries.shape
    top_vals, top_idx = _run_topk(queries, keys)

    info = plsc.get_sparse_core_info()
    agg = _make_aggregate_sc(q_n, d, info.num_cores, info.num_subcores,
                             info.num_lanes)
    out = agg(top_idx.reshape(-1), top_vals.reshape(-1), values, confidence)
    return out


# speedup vs baseline: 1.1506x; 1.0032x over previous
import functools

import jax
import jax.numpy as jnp
from jax import lax
from jax.experimental import pallas as pl
from jax.experimental.pallas import tpu as pltpu

try:
    from jax.experimental.pallas import tpu_sc as plsc
except ImportError:
    plsc = None

_K = 8
_NEG = -1e30
_FBIG = 3e38


def _topk_body(q_ref, k_ref, vals_ref, idx_ref, cv_scr, ci_scr, *,
               n_keys, tile, nt):
    t = pl.program_id(0)

    q = q_ref[...]
    qn = q / jnp.maximum(jnp.sqrt(jnp.sum(q * q, axis=1, keepdims=True)), 1e-8)
    k = k_ref[...]
    kn = k / jnp.maximum(jnp.sqrt(jnp.sum(k * k, axis=1, keepdims=True)), 1e-8)

    s = lax.dot_general(qn, kn, (((1,), (1,)), ((), ())),
                        preferred_element_type=jnp.float32)
    gcol = (jnp.float32(t * tile)
            + lax.broadcasted_iota(jnp.int32, s.shape, 1).astype(jnp.float32))
    s = jnp.where(gcol < jnp.float32(n_keys), s, _NEG)

    tv, ti = [], []
    for _ in range(_K):
        m = jnp.max(s, axis=1, keepdims=True)
        ci = jnp.min(jnp.where(s >= m, gcol, _FBIG), axis=1, keepdims=True)
        tv.append(m)
        ti.append(ci)
        s = jnp.where(gcol == ci, _NEG, s)
    cv_scr[pl.ds(t * _K, _K), :] = jnp.concatenate(tv, axis=1).T
    ci_scr[pl.ds(t * _K, _K), :] = jnp.concatenate(ti, axis=1).T

    @pl.when(t == nt - 1)
    def _merge():
        mv = cv_scr[...]
        mi = ci_scr[...]
        nv, ni = [], []
        for _ in range(_K):
            m = jnp.max(mv, axis=0, keepdims=True)
            ci = jnp.min(jnp.where(mv >= m, mi, _FBIG), axis=0, keepdims=True)
            nv.append(m)
            ni.append(ci)
            mv = jnp.where(mi == ci, _NEG, mv)
        vals_ref[...] = jnp.concatenate(nv, axis=0)
        idx_ref[...] = jnp.concatenate(ni, axis=0).astype(jnp.int32)


def _run_topk(queries, keys):
    q_n, d = queries.shape
    n_keys = keys.shape[0]
    tile = 1024
    nt = (n_keys + tile - 1) // tile
    return pl.pallas_call(
        functools.partial(_topk_body, n_keys=n_keys, tile=tile, nt=nt),
        grid=(nt,),
        in_specs=[
            pl.BlockSpec((q_n, d), lambda t: (0, 0)),
            pl.BlockSpec((tile, d), lambda t: (t, 0)),
        ],
        out_specs=[
            pl.BlockSpec((_K, q_n), lambda t: (0, 0)),
            pl.BlockSpec((_K, q_n), lambda t: (0, 0)),
        ],
        out_shape=[
            jax.ShapeDtypeStruct((_K, q_n), jnp.float32),
            jax.ShapeDtypeStruct((_K, q_n), jnp.int32),
        ],
        scratch_shapes=[
            pltpu.VMEM((nt * _K, q_n), jnp.float32),
            pltpu.VMEM((nt * _K, q_n), jnp.float32),
        ],
        compiler_params=pltpu.CompilerParams(
            dimension_semantics=("arbitrary",)),
    )(queries, keys)


def _perm_lanes(w, idx):
    lanes = w.shape[0]
    return lax.gather(
        w,
        idx.reshape(lanes, 1),
        lax.GatherDimensionNumbers(
            offset_dims=(), collapsed_slice_dims=(0,), start_index_map=(0,)),
        slice_sizes=(1,),
        mode=lax.GatherScatterMode.PROMISE_IN_BOUNDS,
    )


def _bcast_lane(w, j, lanes):
    return _perm_lanes(w, jnp.full((lanes,), j, jnp.int32))


def _make_aggregate_sc(q_n, d, n_cores, n_subcores, lanes):
    qpw = q_n // (n_cores * n_subcores)
    pairs = qpw // 2
    mesh = plsc.VectorSubcoreMesh(core_axis_name="c", subcore_axis_name="s")
    nvec = d // lanes

    @functools.partial(
        pl.kernel,
        mesh=mesh,
        out_type=jax.ShapeDtypeStruct((q_n, d), jnp.float32),
        scratch_types=[
            pltpu.VMEM((qpw * _K,), jnp.int32),
            pltpu.VMEM((qpw * _K,), jnp.float32),
            pltpu.VMEM((16,), jnp.float32),
            pltpu.VMEM((16, d), jnp.float32),
            pltpu.VMEM((qpw, d), jnp.float32),
            pltpu.SemaphoreType.DMA,
        ],
    )
    def agg(idx_hbm, vals_hbm, values_hbm, conf_hbm, out_hbm,
            idx_v, sims_v, conf_v, rows_v, out_v, sem):
        wid = lax.axis_index("s") * n_cores + lax.axis_index("c")
        base = wid * (qpw * _K)
        pltpu.sync_copy(idx_hbm.at[pl.ds(base, qpw * _K)], idx_v)
        pltpu.sync_copy(vals_hbm.at[pl.ds(base, qpw * _K)], sims_v)

        for p in range(pairs):
            sl = pl.ds(p * 16, 16)
            idx16 = idx_v[sl]
            sim16 = sims_v[sl]
            pltpu.async_copy(conf_hbm.at[idx16], conf_v, sem).wait()
            pltpu.async_copy(values_hbm.at[idx16], rows_v, sem).wait()
            conf16 = conf_v[...]
            w = jnp.maximum(conf16, jnp.float32(1e-4)) * sim16
            lane = lax.iota(jnp.int32, 16)
            gsum = w
            for sh in (4, 2, 1):
                gsum = gsum + _perm_lanes(gsum, lane ^ sh)
            wn = w / (gsum + jnp.float32(1e-8))
            wbc = [_bcast_lane(wn, j, lanes) for j in range(16)]
            for c in range(nvec):
                csl = pl.ds(c * lanes, lanes)
                acc_a = jnp.zeros((lanes,), jnp.float32)
                acc_b = jnp.zeros((lanes,), jnp.float32)
                for j in range(_K):
                    acc_a = acc_a + wbc[j] * rows_v[j, csl]
                for j in range(_K, 16):
                    acc_b = acc_b + wbc[j] * rows_v[j, csl]
                out_v[2 * p, csl] = acc_a
                out_v[2 * p + 1, csl] = acc_b

        pltpu.sync_copy(out_v, out_hbm.at[pl.ds(wid * qpw, qpw)])

    return agg


def kernel(queries, keys, values, confidence, top_k):
    q_n, d = queries.shape
    top_vals, top_idx = _run_topk(queries, keys)

    info = plsc.get_sparse_core_info()
    agg = _make_aggregate_sc(q_n, d, info.num_cores, info.num_subcores,
                             info.num_lanes)
    return agg(top_idx.T.reshape(-1), top_vals.T.reshape(-1),
               values, confidence)


# speedup vs baseline: 1.1681x; 1.0152x over previous
import functools

import jax
import jax.numpy as jnp
from jax import lax
from jax.experimental import pallas as pl
from jax.experimental.pallas import tpu as pltpu

try:
    from jax.experimental.pallas import tpu_sc as plsc
except ImportError:
    plsc = None

_K = 8
_NEG = -1e30
_FBIG = 3e38


def _topk_body(q_ref, k_ref, vals_ref, idx_ref, cv_scr, ci_scr, *,
               n_keys, tile, nt):
    t = pl.program_id(0)

    q = q_ref[...]
    qn = q / jnp.maximum(jnp.sqrt(jnp.sum(q * q, axis=1, keepdims=True)), 1e-8)
    k = k_ref[...]
    kn = k / jnp.maximum(jnp.sqrt(jnp.sum(k * k, axis=1, keepdims=True)), 1e-8)

    s = lax.dot_general(qn, kn, (((1,), (1,)), ((), ())),
                        preferred_element_type=jnp.float32)
    gcol = (jnp.float32(t * tile)
            + lax.broadcasted_iota(jnp.int32, s.shape, 1).astype(jnp.float32))
    s = jnp.where(gcol < jnp.float32(n_keys), s, _NEG)

    tv, ti = [], []
    for _ in range(_K):
        m = jnp.max(s, axis=1, keepdims=True)
        ci = jnp.min(jnp.where(s >= m, gcol, _FBIG), axis=1, keepdims=True)
        tv.append(m)
        ti.append(ci)
        s = jnp.where(gcol == ci, _NEG, s)
    cv_scr[pl.ds(t * _K, _K), :] = jnp.concatenate(tv, axis=1).T
    ci_scr[pl.ds(t * _K, _K), :] = jnp.concatenate(ti, axis=1).T

    @pl.when(t == nt - 1)
    def _merge():
        mv = cv_scr[...]
        mi = ci_scr[...]
        nv, ni = [], []
        for _ in range(_K):
            m = jnp.max(mv, axis=0, keepdims=True)
            ci = jnp.min(jnp.where(mv >= m, mi, _FBIG), axis=0, keepdims=True)
            nv.append(m)
            ni.append(ci)
            mv = jnp.where(mi == ci, _NEG, mv)
        vals_ref[...] = jnp.concatenate(nv, axis=0)
        idx_ref[...] = jnp.concatenate(ni, axis=0).astype(jnp.int32)


def _run_topk(queries, keys):
    q_n, d = queries.shape
    n_keys = keys.shape[0]
    tile = 2048
    nt = (n_keys + tile - 1) // tile
    return pl.pallas_call(
        functools.partial(_topk_body, n_keys=n_keys, tile=tile, nt=nt),
        grid=(nt,),
        in_specs=[
            pl.BlockSpec((q_n, d), lambda t: (0, 0)),
            pl.BlockSpec((tile, d), lambda t: (t, 0)),
        ],
        out_specs=[
            pl.BlockSpec((_K, q_n), lambda t: (0, 0)),
            pl.BlockSpec((_K, q_n), lambda t: (0, 0)),
        ],
        out_shape=[
            jax.ShapeDtypeStruct((_K, q_n), jnp.float32),
            jax.ShapeDtypeStruct((_K, q_n), jnp.int32),
        ],
        scratch_shapes=[
            pltpu.VMEM((nt * _K, q_n), jnp.float32),
            pltpu.VMEM((nt * _K, q_n), jnp.float32),
        ],
        compiler_params=pltpu.CompilerParams(
            dimension_semantics=("arbitrary",)),
    )(queries, keys)


def _perm_lanes(w, idx):
    lanes = w.shape[0]
    return lax.gather(
        w,
        idx.reshape(lanes, 1),
        lax.GatherDimensionNumbers(
            offset_dims=(), collapsed_slice_dims=(0,), start_index_map=(0,)),
        slice_sizes=(1,),
        mode=lax.GatherScatterMode.PROMISE_IN_BOUNDS,
    )


def _bcast_lane(w, j, lanes):
    return _perm_lanes(w, jnp.full((lanes,), j, jnp.int32))


def _make_aggregate_sc(q_n, d, n_cores, n_subcores, lanes):
    qpw = q_n // (n_cores * n_subcores)
    pairs = qpw // 2
    mesh = plsc.VectorSubcoreMesh(core_axis_name="c", subcore_axis_name="s")
    nvec = d // lanes

    @functools.partial(
        pl.kernel,
        mesh=mesh,
        out_type=jax.ShapeDtypeStruct((q_n, d), jnp.float32),
        scratch_types=[
            pltpu.VMEM((qpw * _K,), jnp.int32),
            pltpu.VMEM((qpw * _K,), jnp.float32),
            pltpu.VMEM((16,), jnp.float32),
            pltpu.VMEM((16, d), jnp.float32),
            pltpu.VMEM((qpw, d), jnp.float32),
            pltpu.SemaphoreType.DMA,
        ],
    )
    def agg(idx_hbm, vals_hbm, values_hbm, conf_hbm, out_hbm,
            idx_v, sims_v, conf_v, rows_v, out_v, sem):
        wid = lax.axis_index("s") * n_cores + lax.axis_index("c")
        base = wid * (qpw * _K)
        pltpu.sync_copy(idx_hbm.at[pl.ds(base, qpw * _K)], idx_v)
        pltpu.sync_copy(vals_hbm.at[pl.ds(base, qpw * _K)], sims_v)

        for p in range(pairs):
            sl = pl.ds(p * 16, 16)
            idx16 = idx_v[sl]
            sim16 = sims_v[sl]
            pltpu.async_copy(conf_hbm.at[idx16], conf_v, sem).wait()
            pltpu.async_copy(values_hbm.at[idx16], rows_v, sem).wait()
            conf16 = conf_v[...]
            w = jnp.maximum(conf16, jnp.float32(1e-4)) * sim16
            lane = lax.iota(jnp.int32, 16)
            gsum = w
            for sh in (4, 2, 1):
                gsum = gsum + _perm_lanes(gsum, lane ^ sh)
            wn = w / (gsum + jnp.float32(1e-8))
            wbc = [_bcast_lane(wn, j, lanes) for j in range(16)]
            for c in range(nvec):
                csl = pl.ds(c * lanes, lanes)
                acc_a = jnp.zeros((lanes,), jnp.float32)
                acc_b = jnp.zeros((lanes,), jnp.float32)
                for j in range(_K):
                    acc_a = acc_a + wbc[j] * rows_v[j, csl]
                for j in range(_K, 16):
                    acc_b = acc_b + wbc[j] * rows_v[j, csl]
                out_v[2 * p, csl] = acc_a
                out_v[2 * p + 1, csl] = acc_b

        pltpu.sync_copy(out_v, out_hbm.at[pl.ds(wid * qpw, qpw)])

    return agg


def kernel(queries, keys, values, confidence, top_k):
    q_n, d = queries.shape
    top_vals, top_idx = _run_topk(queries, keys)

    info = plsc.get_sparse_core_info()
    agg = _make_aggregate_sc(q_n, d, info.num_cores, info.num_subcores,
                             info.num_lanes)
    return agg(top_idx.T.reshape(-1), top_vals.T.reshape(-1),
               values, confidence)
